# baseline (speedup 1.0000x reference)
import functools

import jax
import jax.numpy as jnp
from jax import lax
from jax.experimental import pallas as pl
from jax.experimental.pallas import tpu as pltpu
from jax.experimental.pallas import tpu_sc as plsc

T = 2048
D = 768
E = 8
F = 2048
CAP = 256
NW = 32
TPW = T // NW
HW = TPW // 2
RB = 256
GL = 128


def _router_body(tok_ref, wg_ref, bg_ref, dst_ref, g_ref, cnt_ref):
    i = pl.program_id(0)

    @pl.when(i == 0)
    def _init():
        cnt_ref[...] = jnp.zeros((1, E), jnp.float32)

    logits = lax.dot_general(
        tok_ref[0], wg_ref[...], (((1,), (0,)), ((), ())),
        precision=lax.Precision.DEFAULT,
        preferred_element_type=jnp.float32) + bg_ref[...]
    lmax = jnp.max(logits, axis=1, keepdims=True)
    gate = 1.0 / jnp.sum(jnp.exp(logits - lmax), axis=1, keepdims=True)
    iota_e = lax.broadcasted_iota(jnp.int32, (RB, E), 1)
    idx = jnp.min(jnp.where(logits == lmax, iota_e, E), axis=1, keepdims=True)
    mask = (iota_e == idx).astype(jnp.float32)

    iota_r = lax.broadcasted_iota(jnp.int32, (RB, RB), 0)
    iota_c = lax.broadcasted_iota(jnp.int32, (RB, RB), 1)
    tri = (iota_c < iota_r).astype(jnp.float32)
    run = cnt_ref[...]
    pos = jnp.dot(tri, mask, preferred_element_type=jnp.float32) + run
    cnt_ref[...] = run + jnp.sum(mask, axis=0, keepdims=True)

    pos_i = jnp.sum(pos * mask, axis=1, keepdims=True).astype(jnp.int32)
    kept = pos_i < CAP
    row = lax.broadcasted_iota(jnp.int32, (RB, 1), 0)
    dst = jnp.where(kept, idx * CAP + pos_i, T + (row & (CAP - 1)))
    dst_ref[...] = dst.reshape(RB)
    g_ref[...] = jnp.broadcast_to(gate, (RB, GL))


def _router(inputs, Wg, bg):
    return pl.pallas_call(
        _router_body,
        grid=(T // RB,),
        in_specs=[
            pl.BlockSpec((1, RB, D), lambda i: (0, i, 0)),
            pl.BlockSpec((D, E), lambda i: (0, 0)),
            pl.BlockSpec((1, E), lambda i: (0, 0)),
        ],
        out_specs=(
            pl.BlockSpec((RB,), lambda i: (i,)),
            pl.BlockSpec((RB, GL), lambda i: (i, 0)),
        ),
        out_shape=(
            jax.ShapeDtypeStruct((T,), jnp.int32),
            jax.ShapeDtypeStruct((T, GL), jnp.float32),
        ),
        scratch_shapes=[pltpu.VMEM((1, E), jnp.float32)],
    )(inputs, Wg, bg.reshape(1, E))


def _dispatch(inputs, dst, g16):
    mesh = plsc.VectorSubcoreMesh(core_axis_name="c", subcore_axis_name="s")

    @functools.partial(
        pl.kernel, mesh=mesh,
        out_type=(
            jax.ShapeDtypeStruct((T + CAP, D), jnp.float32),
            jax.ShapeDtypeStruct((T + CAP, GL), jnp.float32),
        ),
        scratch_types=[
            pltpu.VMEM((2, HW), jnp.int32),
            pltpu.VMEM((TPW, D), jnp.float32),
            pltpu.VMEM((TPW, GL), jnp.float32),
            pltpu.SemaphoreType.DMA,
            pltpu.SemaphoreType.DMA,
            pltpu.SemaphoreType.DMA,
            pltpu.SemaphoreType.DMA,
        ],
    )
    def k(tok_hbm, dst_hbm, g_hbm, disp_hbm, gates_hbm,
          idx_v, rows_v, gb_v, s0, s1, s2, s3):
        wid = lax.axis_index("s") * 2 + lax.axis_index("c")
        base = wid * TPW
        ld0 = pltpu.async_copy(tok_hbm.at[0, pl.ds(base, HW)],
                               rows_v.at[pl.ds(0, HW)], s0)
        ld1 = pltpu.async_copy(tok_hbm.at[0, pl.ds(base + HW, HW)],
                               rows_v.at[pl.ds(HW, HW)], s1)
        pltpu.sync_copy(dst_hbm.at[pl.ds(base, HW)], idx_v.at[0])
        pltpu.sync_copy(dst_hbm.at[pl.ds(base + HW, HW)], idx_v.at[1])
        pltpu.sync_copy(g_hbm.at[pl.ds(base, TPW)], gb_v)
        ld0.wait()
        st0 = pltpu.async_copy(rows_v.at[pl.ds(0, HW)],
                               disp_hbm.at[idx_v.at[0]], s2)
        ld1.wait()
        st1 = pltpu.async_copy(rows_v.at[pl.ds(HW, HW)],
                               disp_hbm.at[idx_v.at[1]], s3)
        pltpu.sync_copy(gb_v.at[pl.ds(0, HW)], gates_hbm.at[idx_v.at[0]])
        pltpu.sync_copy(gb_v.at[pl.ds(HW, HW)], gates_hbm.at[idx_v.at[1]])
        st0.wait()
        st1.wait()

    return k(inputs, dst, g16)


def _ffn_body(d_ref, g_ref, w1_ref, b1_ref, w2_ref, b2_ref, o_ref):
    e = pl.program_id(0)

    @pl.when(e == E)
    def _zero():
        o_ref[...] = jnp.zeros((CAP, D), jnp.float32)

    @pl.when(e < E)
    def _compute():
        sel1 = (lax.broadcasted_iota(jnp.int32, (E, F), 0) == e).astype(
            jnp.float32)
        b1row = jnp.sum(b1_ref[...] * sel1, axis=0, keepdims=True)
        sel2 = (lax.broadcasted_iota(jnp.int32, (E, D), 0) == e).astype(
            jnp.float32)
        b2row = jnp.sum(b2_ref[...] * sel2, axis=0, keepdims=True)
        x = d_ref[...]
        h = jnp.dot(x, w1_ref[0], precision=lax.Precision.DEFAULT,
                    preferred_element_type=jnp.float32) + b1row
        h = jnp.maximum(h, 0.0)
        o_ref[...] = (jnp.dot(h, w2_ref[0], precision=lax.Precision.DEFAULT,
                              preferred_element_type=jnp.float32)
                      + b2row) * g_ref[..., 0:1]


def _ffn(disp, gates, W1, b1, W2, b2):
    e7 = lambda e: jnp.minimum(e, E - 1)
    return pl.pallas_call(
        _ffn_body,
        grid=(E + 1,),
        in_specs=[
            pl.BlockSpec((CAP, D), lambda e: (e7(e), 0)),
            pl.BlockSpec((CAP, GL), lambda e: (e7(e), 0)),
            pl.BlockSpec((1, D, F), lambda e: (e7(e), 0, 0)),
            pl.BlockSpec((E, F), lambda e: (0, 0)),
            pl.BlockSpec((1, F, D), lambda e: (e7(e), 0, 0)),
            pl.BlockSpec((E, D), lambda e: (0, 0)),
        ],
        out_specs=pl.BlockSpec((CAP, D), lambda e: (e, 0)),
        out_shape=jax.ShapeDtypeStruct((T + CAP, D), jnp.float32),
    )(disp, gates, W1, b1, W2, b2)


def _combine(eo, dst):
    mesh = plsc.VectorSubcoreMesh(core_axis_name="c", subcore_axis_name="s")

    @functools.partial(
        pl.kernel, mesh=mesh,
        out_type=jax.ShapeDtypeStruct((T, D), jnp.float32),
        scratch_types=[
            pltpu.VMEM((TPW,), jnp.int32),
            pltpu.VMEM((TPW, D), jnp.float32),
            pltpu.SemaphoreType.DMA,
            pltpu.SemaphoreType.DMA,
            pltpu.SemaphoreType.DMA,
            pltpu.SemaphoreType.DMA,
        ],
    )
    def k(eo_hbm, dst_hbm, out_hbm, idx_v, rows_v, s0, s1, s2, s3):
        wid = lax.axis_index("s") * 2 + lax.axis_index("c")
        base = wid * TPW
        pltpu.sync_copy(dst_hbm.at[pl.ds(base, TPW)], idx_v)
        g0 = pltpu.async_copy(eo_hbm.at[idx_v.at[pl.ds(0, HW)]],
                              rows_v.at[pl.ds(0, HW)], s0)
        g1 = pltpu.async_copy(eo_hbm.at[idx_v.at[pl.ds(HW, HW)]],
                              rows_v.at[pl.ds(HW, HW)], s1)
        g0.wait()
        w0 = pltpu.async_copy(rows_v.at[pl.ds(0, HW)],
                              out_hbm.at[pl.ds(base, HW)], s2)
        g1.wait()
        w1 = pltpu.async_copy(rows_v.at[pl.ds(HW, HW)],
                              out_hbm.at[pl.ds(base + HW, HW)], s3)
        w0.wait()
        w1.wait()

    return k(eo, dst)


def kernel(inputs, Wg, bg, W1, b1, W2, b2):
    dst, g16 = _router(inputs, Wg, bg)
    disp, gates = _dispatch(inputs, dst, g16)
    eo = _ffn(disp, gates, W1, b1, W2, b2)
    out = _combine(eo, dst)
    return out.reshape(inputs.shape)


# speedup vs baseline: 1.0272x; 1.0272x over previous
import functools

import jax
import jax.numpy as jnp
from jax import lax
from jax.experimental import pallas as pl
from jax.experimental.pallas import tpu as pltpu
from jax.experimental.pallas import tpu_sc as plsc

T = 2048
D = 768
E = 8
F = 2048
CAP = 256
NW = 32
TPW = T // NW
HW = TPW // 2
RB = 256
GL = 128


def _router_body(tok_ref, wg_ref, bg_ref, dst_ref, g_ref):
    logits = lax.dot_general(
        tok_ref[0], wg_ref[...], (((1,), (0,)), ((), ())),
        precision=lax.Precision.DEFAULT,
        preferred_element_type=jnp.float32) + bg_ref[...]
    lmax = jnp.max(logits, axis=1, keepdims=True)
    gate = 1.0 / jnp.sum(jnp.exp(logits - lmax), axis=1, keepdims=True)
    iota_e = lax.broadcasted_iota(jnp.int32, (T, E), 1)
    idx = jnp.min(jnp.where(logits == lmax, iota_e, E), axis=1, keepdims=True)
    mask = (iota_e == idx).astype(jnp.float32)

    iota_r = lax.broadcasted_iota(jnp.int32, (RB, RB), 0)
    iota_c = lax.broadcasted_iota(jnp.int32, (RB, RB), 1)
    tri = (iota_c < iota_r).astype(jnp.float32)

    nblk = T // RB
    blocks = [lax.slice(mask, (k * RB, 0), ((k + 1) * RB, E)) for k in range(nblk)]
    intra = [jnp.dot(tri, b, preferred_element_type=jnp.float32) for b in blocks]
    csum = [jnp.sum(b, axis=0, keepdims=True) for b in blocks]
    off = jnp.zeros((1, E), jnp.float32)
    pos_blocks = []
    for k in range(nblk):
        pos_blocks.append(intra[k] + off)
        off = off + csum[k]
    pos = jnp.concatenate(pos_blocks, axis=0)

    pos_i = jnp.sum(pos * mask, axis=1, keepdims=True).astype(jnp.int32)
    kept = pos_i < CAP
    row = lax.broadcasted_iota(jnp.int32, (T, 1), 0)
    dst = jnp.where(kept, idx * CAP + pos_i, T + (row & (CAP - 1)))
    dst_ref[...] = dst.reshape(T)
    g_ref[...] = jnp.broadcast_to(gate, (T, GL))


def _router(inputs, Wg, bg):
    return pl.pallas_call(
        _router_body,
        out_shape=(
            jax.ShapeDtypeStruct((T,), jnp.int32),
            jax.ShapeDtypeStruct((T, GL), jnp.float32),
        ),
    )(inputs, Wg, bg.reshape(1, E))


def _dispatch(inputs, dst, g16):
    mesh = plsc.VectorSubcoreMesh(core_axis_name="c", subcore_axis_name="s")

    @functools.partial(
        pl.kernel, mesh=mesh,
        out_type=(
            jax.ShapeDtypeStruct((T + CAP, D), jnp.float32),
            jax.ShapeDtypeStruct((T + CAP, GL), jnp.float32),
        ),
        scratch_types=[
            pltpu.VMEM((2, HW), jnp.int32),
            pltpu.VMEM((TPW, D), jnp.float32),
            pltpu.VMEM((TPW, GL), jnp.float32),
            pltpu.SemaphoreType.DMA,
            pltpu.SemaphoreType.DMA,
            pltpu.SemaphoreType.DMA,
            pltpu.SemaphoreType.DMA,
        ],
    )
    def k(tok_hbm, dst_hbm, g_hbm, disp_hbm, gates_hbm,
          idx_v, rows_v, gb_v, s0, s1, s2, s3):
        wid = lax.axis_index("s") * 2 + lax.axis_index("c")
        base = wid * TPW
        ld0 = pltpu.async_copy(tok_hbm.at[0, pl.ds(base, HW)],
                               rows_v.at[pl.ds(0, HW)], s0)
        ld1 = pltpu.async_copy(tok_hbm.at[0, pl.ds(base + HW, HW)],
                               rows_v.at[pl.ds(HW, HW)], s1)
        pltpu.sync_copy(dst_hbm.at[pl.ds(base, HW)], idx_v.at[0])
        pltpu.sync_copy(dst_hbm.at[pl.ds(base + HW, HW)], idx_v.at[1])
        pltpu.sync_copy(g_hbm.at[pl.ds(base, TPW)], gb_v)
        ld0.wait()
        st0 = pltpu.async_copy(rows_v.at[pl.ds(0, HW)],
                               disp_hbm.at[idx_v.at[0]], s2)
        ld1.wait()
        st1 = pltpu.async_copy(rows_v.at[pl.ds(HW, HW)],
                               disp_hbm.at[idx_v.at[1]], s3)
        pltpu.sync_copy(gb_v.at[pl.ds(0, HW)], gates_hbm.at[idx_v.at[0]])
        pltpu.sync_copy(gb_v.at[pl.ds(HW, HW)], gates_hbm.at[idx_v.at[1]])
        st0.wait()
        st1.wait()

    return k(inputs, dst, g16)


def _ffn_body(d_ref, g_ref, w1_ref, b1_ref, w2_ref, b2_ref, o_ref):
    e = pl.program_id(0)

    @pl.when(e == E)
    def _zero():
        o_ref[...] = jnp.zeros((CAP, D), jnp.float32)

    @pl.when(e < E)
    def _compute():
        sel1 = (lax.broadcasted_iota(jnp.int32, (E, F), 0) == e).astype(
            jnp.float32)
        b1row = jnp.sum(b1_ref[...] * sel1, axis=0, keepdims=True)
        sel2 = (lax.broadcasted_iota(jnp.int32, (E, D), 0) == e).astype(
            jnp.float32)
        b2row = jnp.sum(b2_ref[...] * sel2, axis=0, keepdims=True)
        x = d_ref[...]
        h = jnp.dot(x, w1_ref[0], precision=lax.Precision.DEFAULT,
                    preferred_element_type=jnp.float32) + b1row
        h = jnp.maximum(h, 0.0)
        o_ref[...] = (jnp.dot(h, w2_ref[0], precision=lax.Precision.DEFAULT,
                              preferred_element_type=jnp.float32)
                      + b2row) * g_ref[..., 0:1]


def _ffn(disp, gates, W1, b1, W2, b2):
    e7 = lambda e: jnp.minimum(e, E - 1)
    return pl.pallas_call(
        _ffn_body,
        grid=(E + 1,),
        in_specs=[
            pl.BlockSpec((CAP, D), lambda e: (e7(e), 0)),
            pl.BlockSpec((CAP, GL), lambda e: (e7(e), 0)),
            pl.BlockSpec((1, D, F), lambda e: (e7(e), 0, 0)),
            pl.BlockSpec((E, F), lambda e: (0, 0)),
            pl.BlockSpec((1, F, D), lambda e: (e7(e), 0, 0)),
            pl.BlockSpec((E, D), lambda e: (0, 0)),
        ],
        out_specs=pl.BlockSpec((CAP, D), lambda e: (e, 0)),
        out_shape=jax.ShapeDtypeStruct((T + CAP, D), jnp.float32),
    )(disp, gates, W1, b1, W2, b2)


def _combine(eo, dst):
    mesh = plsc.VectorSubcoreMesh(core_axis_name="c", subcore_axis_name="s")

    @functools.partial(
        pl.kernel, mesh=mesh,
        out_type=jax.ShapeDtypeStruct((T, D), jnp.float32),
        scratch_types=[
            pltpu.VMEM((TPW,), jnp.int32),
            pltpu.VMEM((TPW, D), jnp.float32),
            pltpu.SemaphoreType.DMA,
            pltpu.SemaphoreType.DMA,
            pltpu.SemaphoreType.DMA,
            pltpu.SemaphoreType.DMA,
        ],
    )
    def k(eo_hbm, dst_hbm, out_hbm, idx_v, rows_v, s0, s1, s2, s3):
        wid = lax.axis_index("s") * 2 + lax.axis_index("c")
        base = wid * TPW
        pltpu.sync_copy(dst_hbm.at[pl.ds(base, TPW)], idx_v)
        g0 = pltpu.async_copy(eo_hbm.at[idx_v.at[pl.ds(0, HW)]],
                              rows_v.at[pl.ds(0, HW)], s0)
        g1 = pltpu.async_copy(eo_hbm.at[idx_v.at[pl.ds(HW, HW)]],
                              rows_v.at[pl.ds(HW, HW)], s1)
        g0.wait()
        w0 = pltpu.async_copy(rows_v.at[pl.ds(0, HW)],
                              out_hbm.at[pl.ds(base, HW)], s2)
        g1.wait()
        w1 = pltpu.async_copy(rows_v.at[pl.ds(HW, HW)],
                              out_hbm.at[pl.ds(base + HW, HW)], s3)
        w0.wait()
        w1.wait()

    return k(eo, dst)


def kernel(inputs, Wg, bg, W1, b1, W2, b2):
    dst, g16 = _router(inputs, Wg, bg)
    disp, gates = _dispatch(inputs, dst, g16)
    eo = _ffn(disp, gates, W1, b1, W2, b2)
    out = _combine(eo, dst)
    return out.reshape(inputs.shape)
